# baseline (speedup 1.0000x reference)
import jax
import jax.numpy as jnp
from jax import lax
from jax.experimental import pallas as pl
from jax.experimental.pallas import tpu as pltpu
from jax.experimental.pallas import tpu_sc as plsc

_B = 16384
_D = 64
_NC = 2
_NS = 16
_NW = _NC * _NS
_BPW = _B // _NW
_CH = 128
_NCH = _BPW // _CH
_G = 16
_NG = _CH // _G


def _sc_body(user_id, film_id, user_table, film_table, user_bias, film_bias,
             out, idx_u, idx_f, row_u, row_f, col_u, col_f,
             u_rows, f_rows, ub_v, fb_v, out_v, sem):
    wid = lax.axis_index("s") * _NC + lax.axis_index("c")
    base = wid * _BPW

    pltpu.sync_copy(user_id.at[pl.ds(base, _BPW)], idx_u)
    pltpu.sync_copy(film_id.at[pl.ds(base, _BPW)], idx_f)

    def precompute(i, carry):
        sl = pl.ds(i * _G, _G)
        iu = idx_u[sl]
        iff = idx_f[sl]
        row_u[sl] = iu >> 1
        row_f[sl] = iff >> 1
        col_u[sl] = (iu & 1) << 6
        col_f[sl] = (iff & 1) << 6
        return carry

    lax.fori_loop(0, _BPW // _G, precompute, 0)

    for c in range(_NCH):
        sl = pl.ds(c * _CH, _CH)
        cp = [
            pltpu.async_copy(user_table.at[row_u.at[sl]], u_rows, sem),
            pltpu.async_copy(film_table.at[row_f.at[sl]], f_rows, sem),
            pltpu.async_copy(user_bias.at[idx_u.at[sl]], ub_v, sem),
            pltpu.async_copy(film_bias.at[idx_f.at[sl]], fb_v, sem),
        ]
        for h in cp:
            h.wait()

        def group(g, carry):
            rows = g * _G + lax.iota(jnp.int32, _G)
            sl16 = pl.ds(c * _CH + g * _G, _G)
            cb_u = col_u[sl16]
            cb_f = col_f[sl16]
            rot = lax.iota(jnp.int32, _G)
            acc = jnp.zeros((_G,), jnp.float32)
            for d in range(_D):
                cu = cb_u + rot
                cf = cb_f + rot
                uu = plsc.load_gather(u_rows, [rows, cu])
                ff = plsc.load_gather(f_rows, [rows, cf])
                acc = acc + uu * ff
                rot = (rot + 1) & (_D - 1)
            slg = pl.ds(g * _G, _G)
            out_v[sl16] = acc + ub_v[slg] + fb_v[slg]
            return carry

        lax.fori_loop(0, _NG, group, 0)

    pltpu.sync_copy(out_v, out.at[pl.ds(base, _BPW)])


@jax.jit
def _run(user_id, film_id, user_table, film_table, user_bias, film_bias):
    mesh = plsc.VectorSubcoreMesh(core_axis_name="c", subcore_axis_name="s")
    f = pl.kernel(
        _sc_body,
        out_type=jax.ShapeDtypeStruct((_B,), jnp.float32),
        mesh=mesh,
        compiler_params=pltpu.CompilerParams(needs_layout_passes=False),
        scratch_types=[
            pltpu.VMEM((_BPW,), jnp.int32),
            pltpu.VMEM((_BPW,), jnp.int32),
            pltpu.VMEM((_BPW,), jnp.int32),
            pltpu.VMEM((_BPW,), jnp.int32),
            pltpu.VMEM((_BPW,), jnp.int32),
            pltpu.VMEM((_BPW,), jnp.int32),
            pltpu.VMEM((_CH, 2 * _D), jnp.float32),
            pltpu.VMEM((_CH, 2 * _D), jnp.float32),
            pltpu.VMEM((_CH,), jnp.float32),
            pltpu.VMEM((_CH,), jnp.float32),
            pltpu.VMEM((_BPW,), jnp.float32),
            pltpu.SemaphoreType.DMA,
        ],
    )
    return f(user_id, film_id, user_table, film_table, user_bias, film_bias)


def kernel(user_id, film_id, user_table, film_table, user_bias_table,
           film_bias_table):
    ut = user_table.reshape((-1, 2 * _D))
    ft = film_table.reshape((-1, 2 * _D))
    ub = user_bias_table.reshape((-1,))
    fb = film_bias_table.reshape((-1,))
    return _run(user_id, film_id, ut, ft, ub, fb)
